# baseline (speedup 1.0000x reference)
import jax
import jax.numpy as jnp
import numpy as np
from jax.experimental import pallas as pl
from jax.experimental.pallas import tpu as pltpu

_N_EDGE = 1600000
_D_E = 16
_N_NODE = 50000
_GRID = 20
_CW = _N_EDGE // _GRID
_NSTEPS = 5
_NBLK = _N_NODE // _NSTEPS


def _body(g_ref, e_ref, n_ref, w_ref, b_ref, o_ref, acc_e, acc_n):
    step = pl.program_id(0)

    @pl.when(step == 0)
    def _init():
        acc_e[...] = jnp.zeros_like(acc_e)
        acc_n[...] = jnp.zeros_like(acc_n)

    acc_e[...] += e_ref[...]

    @pl.when(step < _NSTEPS)
    def _node_phase():
        acc_n[...] += jnp.sum(n_ref[...].reshape(_NBLK // 8, 8, 128), axis=0)

    @pl.when(step == pl.num_programs(0) - 1)
    def _finish():
        e_sum = jnp.sum(acc_e[...], axis=1, keepdims=True)
        dn = (((1,), (0,)), ((), ()))
        out = jax.lax.dot_general(
            g_ref[...], w_ref[0:128, :], dn,
            preferred_element_type=jnp.float32,
        )
        out += jax.lax.dot_general(
            e_sum * (1.0 / _N_EDGE), w_ref[128:144, :],
            (((0,), (0,)), ((), ())),
            preferred_element_type=jnp.float32,
        )
        n_sum = jnp.sum(acc_n[...], axis=0, keepdims=True)
        out += jax.lax.dot_general(
            n_sum * (1.0 / _N_NODE), w_ref[144:272, :], dn,
            preferred_element_type=jnp.float32,
        )
        o_ref[...] = out + b_ref[...]


@jax.jit
def kernel(global_attr, edge_attr, node_attr, W, b):
    e_t = edge_attr.T
    g_row = global_attr.reshape(1, 128)
    b_row = b.reshape(1, 128)

    out_row = pl.pallas_call(
        _body,
        grid=(_GRID,),
        in_specs=[
            pl.BlockSpec((1, 128), lambda i: (0, 0)),
            pl.BlockSpec((_D_E, _CW), lambda i: (0, i)),
            pl.BlockSpec((_NBLK, 128), lambda i: (jnp.minimum(i, _NSTEPS - 1), 0)),
            pl.BlockSpec((272, 128), lambda i: (0, 0)),
            pl.BlockSpec((1, 128), lambda i: (0, 0)),
        ],
        out_specs=pl.BlockSpec((1, 128), lambda i: (0, 0)),
        out_shape=jax.ShapeDtypeStruct((1, 128), jnp.float32),
        scratch_shapes=[
            pltpu.VMEM((_D_E, _CW), jnp.float32),
            pltpu.VMEM((8, 128), jnp.float32),
        ],
    )(g_row, e_t, node_attr, W, b_row)
    return out_row.reshape(128)


# speedup vs baseline: 1.0040x; 1.0040x over previous
import jax
import jax.numpy as jnp
import numpy as np
from jax.experimental import pallas as pl
from jax.experimental.pallas import tpu as pltpu

_N_EDGE = 1600000
_D_E = 16
_N_NODE = 50000
_GRID = 10
_CW = _N_EDGE // _GRID
_NSTEPS = 5
_NBLK = _N_NODE // _NSTEPS


def _body(g_ref, e_ref, n_ref, w_ref, b_ref, o_ref, acc_e, acc_n):
    step = pl.program_id(0)

    @pl.when(step == 0)
    def _init():
        acc_e[...] = jnp.zeros_like(acc_e)
        acc_n[...] = jnp.zeros_like(acc_n)

    acc_e[...] += e_ref[...]

    @pl.when(step < _NSTEPS)
    def _node_phase():
        acc_n[...] += jnp.sum(n_ref[...].reshape(_NBLK // 8, 8, 128), axis=0)

    @pl.when(step == pl.num_programs(0) - 1)
    def _finish():
        e_sum = jnp.sum(acc_e[...], axis=1, keepdims=True)
        dn = (((1,), (0,)), ((), ()))
        out = jax.lax.dot_general(
            g_ref[...], w_ref[0:128, :], dn,
            preferred_element_type=jnp.float32,
        )
        out += jax.lax.dot_general(
            e_sum * (1.0 / _N_EDGE), w_ref[128:144, :],
            (((0,), (0,)), ((), ())),
            preferred_element_type=jnp.float32,
        )
        n_sum = jnp.sum(acc_n[...], axis=0, keepdims=True)
        out += jax.lax.dot_general(
            n_sum * (1.0 / _N_NODE), w_ref[144:272, :], dn,
            preferred_element_type=jnp.float32,
        )
        o_ref[...] = out + b_ref[...]


@jax.jit
def kernel(global_attr, edge_attr, node_attr, W, b):
    e_t = edge_attr.T
    g_row = global_attr.reshape(1, 128)
    b_row = b.reshape(1, 128)

    out_row = pl.pallas_call(
        _body,
        grid=(_GRID,),
        in_specs=[
            pl.BlockSpec((1, 128), lambda i: (0, 0)),
            pl.BlockSpec((_D_E, _CW), lambda i: (0, i)),
            pl.BlockSpec((_NBLK, 128), lambda i: (jnp.minimum(i, _NSTEPS - 1), 0)),
            pl.BlockSpec((272, 128), lambda i: (0, 0)),
            pl.BlockSpec((1, 128), lambda i: (0, 0)),
        ],
        out_specs=pl.BlockSpec((1, 128), lambda i: (0, 0)),
        out_shape=jax.ShapeDtypeStruct((1, 128), jnp.float32),
        scratch_shapes=[
            pltpu.VMEM((_D_E, _CW), jnp.float32),
            pltpu.VMEM((8, 128), jnp.float32),
        ],
    )(g_row, e_t, node_attr, W, b_row)
    return out_row.reshape(128)
